# baseline (speedup 1.0000x reference)
import jax
import jax.numpy as jnp
from jax.experimental import pallas as pl

_B = 64
_RATIO = 0.2
_C = 512
_CR = 256


def _score_count_kernel(logits_ref, proj_ref, bcol_ref, scol_ref, cnt_ref):
    i = pl.program_id(0)
    proj = proj_ref[...]
    norm = jnp.sqrt(jnp.sum(proj * proj))
    lg = logits_ref[...]
    raw = jax.lax.dot_general(lg, proj, (((1,), (1,)), ((), ())),
                              preferred_element_type=jnp.float32,
                              precision=jax.lax.Precision.HIGHEST)
    scol_ref[...] = jax.nn.sigmoid(raw / norm)

    @pl.when(i == 0)
    def _():
        cnt_ref[...] = jnp.zeros_like(cnt_ref)

    b = bcol_ref[...]
    gid = jax.lax.broadcasted_iota(jnp.int32, (1, _B), 1)
    eq = (b == gid).astype(jnp.float32)
    cnt_ref[...] += jnp.sum(eq, axis=0, keepdims=True)


def _rank_kernel(scol_ref, bcol_ref, srow_ref, brow_ref, rank_ref):
    i = pl.program_id(0)
    c = scol_ref.shape[0]
    s_i = scol_ref[...]
    b_i = bcol_ref[...]
    idx_i = i * c + jax.lax.broadcasted_iota(jnp.int32, (c, 1), 0)
    bmin_i = b_i[0, 0]
    bmax_i = b_i[c - 1, 0]
    bfirst = brow_ref[:, 0:1]
    blast = brow_ref[:, c - 1:c]
    j_lo = jnp.sum((blast < bmin_i).astype(jnp.int32))
    j_hi = jnp.sum((bfirst <= bmax_i).astype(jnp.int32))

    def body(j, acc):
        s_j = srow_ref[pl.ds(j, 1), :]
        b_j = brow_ref[pl.ds(j, 1), :]
        idx_j = j * c + jax.lax.broadcasted_iota(jnp.int32, (1, c), 1)
        same = b_i == b_j
        ahead = (s_j > s_i) | ((s_j == s_i) & (idx_j < idx_i))
        contrib = jnp.sum(jnp.where(same & ahead, 1.0, 0.0),
                          axis=1, keepdims=True)
        return acc + contrib

    rank_ref[...] = jax.lax.fori_loop(j_lo, j_hi, body,
                                      jnp.zeros((c, 1), jnp.float32))


def _pool_kernel(rank_ref, bcol_ref, counts_ref, logits_ref, out_ref):
    i = pl.program_id(0)

    @pl.when(i == 0)
    def _():
        out_ref[...] = jnp.zeros_like(out_ref)

    k = jnp.ceil(jnp.float32(_RATIO) * counts_ref[...])
    b = bcol_ref[...]
    gid = jax.lax.broadcasted_iota(jnp.int32, (1, _B), 1)
    onehot = (b == gid).astype(jnp.float32)
    k_sel = jax.lax.dot_general(onehot, k, (((1,), (1,)), ((), ())),
                                preferred_element_type=jnp.float32,
                              precision=jax.lax.Precision.HIGHEST)
    keep = (rank_ref[...] < k_sel).astype(jnp.float32)
    m = onehot * keep
    out_ref[...] += jax.lax.dot_general(
        m, logits_ref[...], (((0,), (0,)), ((), ())),
        preferred_element_type=jnp.float32,
                              precision=jax.lax.Precision.HIGHEST)


def kernel(logits, batch, proj):
    n, d = logits.shape
    batch = batch.astype(jnp.int32)
    nch = -(-n // _C)
    npad = nch * _C
    pad = npad - n
    logits_p = jnp.pad(logits, ((0, pad), (0, 0)))
    batch_p = jnp.pad(batch, (0, pad), constant_values=_B)
    bcol = batch_p.reshape(npad, 1)

    scol, counts = pl.pallas_call(
        _score_count_kernel,
        grid=(nch,),
        in_specs=[
            pl.BlockSpec((_C, d), lambda i: (i, 0)),
            pl.BlockSpec((1, d), lambda i: (0, 0)),
            pl.BlockSpec((_C, 1), lambda i: (i, 0)),
        ],
        out_specs=[
            pl.BlockSpec((_C, 1), lambda i: (i, 0)),
            pl.BlockSpec((1, _B), lambda i: (0, 0)),
        ],
        out_shape=[
            jax.ShapeDtypeStruct((npad, 1), jnp.float32),
            jax.ShapeDtypeStruct((1, _B), jnp.float32),
        ],
    )(logits_p, proj, bcol)

    nchr = npad // _CR
    srow = scol.reshape(nchr, _CR)
    brow_r = batch_p.reshape(nchr, _CR)

    rank = pl.pallas_call(
        _rank_kernel,
        grid=(nchr,),
        in_specs=[
            pl.BlockSpec((_CR, 1), lambda i: (i, 0)),
            pl.BlockSpec((_CR, 1), lambda i: (i, 0)),
            pl.BlockSpec((nchr, _CR), lambda i: (0, 0)),
            pl.BlockSpec((nchr, _CR), lambda i: (0, 0)),
        ],
        out_specs=pl.BlockSpec((_CR, 1), lambda i: (i, 0)),
        out_shape=jax.ShapeDtypeStruct((npad, 1), jnp.float32),
    )(scol, bcol, srow, brow_r)

    out = pl.pallas_call(
        _pool_kernel,
        grid=(nch,),
        in_specs=[
            pl.BlockSpec((_C, 1), lambda i: (i, 0)),
            pl.BlockSpec((_C, 1), lambda i: (i, 0)),
            pl.BlockSpec((1, _B), lambda i: (0, 0)),
            pl.BlockSpec((_C, d), lambda i: (i, 0)),
        ],
        out_specs=pl.BlockSpec((_B, d), lambda i: (0, 0)),
        out_shape=jax.ShapeDtypeStruct((_B, d), jnp.float32),
    )(rank, bcol, counts, logits_p)

    return out


# speedup vs baseline: 1.0390x; 1.0390x over previous
import jax
import jax.numpy as jnp
from jax.experimental import pallas as pl

_B = 64
_RATIO = 0.2
_C = 512
_CR = 1024


def _score_count_kernel(logits_ref, proj_ref, bcol_ref, scol_ref, cnt_ref):
    i = pl.program_id(0)
    proj = proj_ref[...]
    norm = jnp.sqrt(jnp.sum(proj * proj))
    lg = logits_ref[...]
    raw = jax.lax.dot_general(lg, proj, (((1,), (1,)), ((), ())),
                              preferred_element_type=jnp.float32,
                              precision=jax.lax.Precision.HIGHEST)
    scol_ref[...] = jax.nn.sigmoid(raw / norm)

    @pl.when(i == 0)
    def _():
        cnt_ref[...] = jnp.zeros_like(cnt_ref)

    b = bcol_ref[...]
    gid = jax.lax.broadcasted_iota(jnp.int32, (1, _B), 1)
    eq = (b == gid).astype(jnp.float32)
    cnt_ref[...] += jnp.sum(eq, axis=0, keepdims=True)


def _rank_kernel(scol_ref, bcol_ref, srow_ref, brow_ref, rank_ref):
    i = pl.program_id(0)
    c = scol_ref.shape[0]
    s_i = scol_ref[...]
    b_i = bcol_ref[...]
    idx_i = i * c + jax.lax.broadcasted_iota(jnp.int32, (c, 1), 0)
    bmin_i = b_i[0, 0]
    bmax_i = b_i[c - 1, 0]
    bfirst = brow_ref[:, 0:1]
    blast = brow_ref[:, c - 1:c]
    j_lo = jnp.sum((blast < bmin_i).astype(jnp.int32))
    j_hi = jnp.sum((bfirst <= bmax_i).astype(jnp.int32))

    def body(j, acc):
        s_j = srow_ref[pl.ds(j, 1), :]
        b_j = brow_ref[pl.ds(j, 1), :]
        idx_j = j * c + jax.lax.broadcasted_iota(jnp.int32, (1, c), 1)
        same = b_i == b_j
        ahead = (s_j > s_i) | ((s_j == s_i) & (idx_j < idx_i))
        contrib = jnp.sum(jnp.where(same & ahead, 1.0, 0.0),
                          axis=1, keepdims=True)
        return acc + contrib

    rank_ref[...] = jax.lax.fori_loop(j_lo, j_hi, body,
                                      jnp.zeros((c, 1), jnp.float32))


def _pool_kernel(rank_ref, bcol_ref, counts_ref, logits_ref, out_ref):
    i = pl.program_id(0)

    @pl.when(i == 0)
    def _():
        out_ref[...] = jnp.zeros_like(out_ref)

    k = jnp.ceil(jnp.float32(_RATIO) * counts_ref[...])
    b = bcol_ref[...]
    gid = jax.lax.broadcasted_iota(jnp.int32, (1, _B), 1)
    onehot = (b == gid).astype(jnp.float32)
    k_sel = jax.lax.dot_general(onehot, k, (((1,), (1,)), ((), ())),
                                preferred_element_type=jnp.float32,
                              precision=jax.lax.Precision.HIGHEST)
    keep = (rank_ref[...] < k_sel).astype(jnp.float32)
    m = onehot * keep
    out_ref[...] += jax.lax.dot_general(
        m, logits_ref[...], (((0,), (0,)), ((), ())),
        preferred_element_type=jnp.float32,
                              precision=jax.lax.Precision.HIGHEST)


def kernel(logits, batch, proj):
    n, d = logits.shape
    batch = batch.astype(jnp.int32)
    nch = -(-n // _C)
    npad = nch * _C
    pad = npad - n
    logits_p = jnp.pad(logits, ((0, pad), (0, 0)))
    batch_p = jnp.pad(batch, (0, pad), constant_values=_B)
    bcol = batch_p.reshape(npad, 1)

    scol, counts = pl.pallas_call(
        _score_count_kernel,
        grid=(nch,),
        in_specs=[
            pl.BlockSpec((_C, d), lambda i: (i, 0)),
            pl.BlockSpec((1, d), lambda i: (0, 0)),
            pl.BlockSpec((_C, 1), lambda i: (i, 0)),
        ],
        out_specs=[
            pl.BlockSpec((_C, 1), lambda i: (i, 0)),
            pl.BlockSpec((1, _B), lambda i: (0, 0)),
        ],
        out_shape=[
            jax.ShapeDtypeStruct((npad, 1), jnp.float32),
            jax.ShapeDtypeStruct((1, _B), jnp.float32),
        ],
    )(logits_p, proj, bcol)

    nchr = npad // _CR
    srow = scol.reshape(nchr, _CR)
    brow_r = batch_p.reshape(nchr, _CR)

    rank = pl.pallas_call(
        _rank_kernel,
        grid=(nchr,),
        in_specs=[
            pl.BlockSpec((_CR, 1), lambda i: (i, 0)),
            pl.BlockSpec((_CR, 1), lambda i: (i, 0)),
            pl.BlockSpec((nchr, _CR), lambda i: (0, 0)),
            pl.BlockSpec((nchr, _CR), lambda i: (0, 0)),
        ],
        out_specs=pl.BlockSpec((_CR, 1), lambda i: (i, 0)),
        out_shape=jax.ShapeDtypeStruct((npad, 1), jnp.float32),
    )(scol, bcol, srow, brow_r)

    out = pl.pallas_call(
        _pool_kernel,
        grid=(nch,),
        in_specs=[
            pl.BlockSpec((_C, 1), lambda i: (i, 0)),
            pl.BlockSpec((_C, 1), lambda i: (i, 0)),
            pl.BlockSpec((1, _B), lambda i: (0, 0)),
            pl.BlockSpec((_C, d), lambda i: (i, 0)),
        ],
        out_specs=pl.BlockSpec((_B, d), lambda i: (0, 0)),
        out_shape=jax.ShapeDtypeStruct((_B, d), jnp.float32),
    )(rank, bcol, counts, logits_p)

    return out


# speedup vs baseline: 1.1550x; 1.1116x over previous
import jax
import jax.numpy as jnp
from jax.experimental import pallas as pl

_B = 64
_RATIO = 0.2
_C = 512
_CR = 512


def _score_count_kernel(logits_ref, proj_ref, bcol_ref, scol_ref, cnt_ref):
    i = pl.program_id(0)
    proj = proj_ref[...]
    norm = jnp.sqrt(jnp.sum(proj * proj))
    lg = logits_ref[...]
    raw = jax.lax.dot_general(lg, proj, (((1,), (1,)), ((), ())),
                              preferred_element_type=jnp.float32,
                              precision=jax.lax.Precision.HIGHEST)
    scol_ref[...] = jax.nn.sigmoid(raw / norm)

    @pl.when(i == 0)
    def _():
        cnt_ref[...] = jnp.zeros_like(cnt_ref)

    b = bcol_ref[...]
    gid = jax.lax.broadcasted_iota(jnp.int32, (1, _B), 1)
    eq = (b == gid).astype(jnp.float32)
    cnt_ref[...] += jnp.sum(eq, axis=0, keepdims=True)


def _rank_kernel(scol_ref, bcol_ref, srow_ref, brow_ref, rank_ref):
    i = pl.program_id(0)
    c = scol_ref.shape[0]
    s_i = scol_ref[...]
    b_i = bcol_ref[...]
    idx_i = i * c + jax.lax.broadcasted_iota(jnp.int32, (c, 1), 0)
    bmin_i = b_i[0, 0]
    bmax_i = b_i[c - 1, 0]
    bfirst = brow_ref[:, 0:1]
    blast = brow_ref[:, c - 1:c]
    j_lo = jnp.sum((blast < bmin_i).astype(jnp.int32))
    j_hi = jnp.sum((bfirst <= bmax_i).astype(jnp.int32))

    def body(j, acc):
        s_j = srow_ref[pl.ds(j, 1), :]
        b_j = brow_ref[pl.ds(j, 1), :]
        idx_j = j * c + jax.lax.broadcasted_iota(jnp.int32, (1, c), 1)
        same = b_i == b_j
        ahead = (s_j > s_i) | ((s_j == s_i) & (idx_j < idx_i))
        contrib = jnp.sum(jnp.where(same & ahead, 1.0, 0.0),
                          axis=1, keepdims=True)
        return acc + contrib

    rank_ref[...] = jax.lax.fori_loop(j_lo, j_hi, body,
                                      jnp.zeros((c, 1), jnp.float32))


def _pool_kernel(rank_ref, bcol_ref, counts_ref, logits_ref, out_ref):
    i = pl.program_id(0)

    @pl.when(i == 0)
    def _():
        out_ref[...] = jnp.zeros_like(out_ref)

    k = jnp.ceil(jnp.float32(_RATIO) * counts_ref[...])
    b = bcol_ref[...]
    gid = jax.lax.broadcasted_iota(jnp.int32, (1, _B), 1)
    onehot = (b == gid).astype(jnp.float32)
    k_sel = jax.lax.dot_general(onehot, k, (((1,), (1,)), ((), ())),
                                preferred_element_type=jnp.float32,
                              precision=jax.lax.Precision.HIGHEST)
    keep = (rank_ref[...] < k_sel).astype(jnp.float32)
    m = onehot * keep
    out_ref[...] += jax.lax.dot_general(
        m, logits_ref[...], (((0,), (0,)), ((), ())),
        preferred_element_type=jnp.float32,
                              precision=jax.lax.Precision.HIGHEST)


def kernel(logits, batch, proj):
    n, d = logits.shape
    batch = batch.astype(jnp.int32)
    nch = -(-n // _C)
    npad = nch * _C
    pad = npad - n
    logits_p = jnp.pad(logits, ((0, pad), (0, 0)))
    batch_p = jnp.pad(batch, (0, pad), constant_values=_B)
    bcol = batch_p.reshape(npad, 1)

    scol, counts = pl.pallas_call(
        _score_count_kernel,
        grid=(nch,),
        in_specs=[
            pl.BlockSpec((_C, d), lambda i: (i, 0)),
            pl.BlockSpec((1, d), lambda i: (0, 0)),
            pl.BlockSpec((_C, 1), lambda i: (i, 0)),
        ],
        out_specs=[
            pl.BlockSpec((_C, 1), lambda i: (i, 0)),
            pl.BlockSpec((1, _B), lambda i: (0, 0)),
        ],
        out_shape=[
            jax.ShapeDtypeStruct((npad, 1), jnp.float32),
            jax.ShapeDtypeStruct((1, _B), jnp.float32),
        ],
    )(logits_p, proj, bcol)

    nchr = npad // _CR
    srow = scol.reshape(nchr, _CR)
    brow_r = batch_p.reshape(nchr, _CR)

    rank = pl.pallas_call(
        _rank_kernel,
        grid=(nchr,),
        in_specs=[
            pl.BlockSpec((_CR, 1), lambda i: (i, 0)),
            pl.BlockSpec((_CR, 1), lambda i: (i, 0)),
            pl.BlockSpec((nchr, _CR), lambda i: (0, 0)),
            pl.BlockSpec((nchr, _CR), lambda i: (0, 0)),
        ],
        out_specs=pl.BlockSpec((_CR, 1), lambda i: (i, 0)),
        out_shape=jax.ShapeDtypeStruct((npad, 1), jnp.float32),
    )(scol, bcol, srow, brow_r)

    out = pl.pallas_call(
        _pool_kernel,
        grid=(nch,),
        in_specs=[
            pl.BlockSpec((_C, 1), lambda i: (i, 0)),
            pl.BlockSpec((_C, 1), lambda i: (i, 0)),
            pl.BlockSpec((1, _B), lambda i: (0, 0)),
            pl.BlockSpec((_C, d), lambda i: (i, 0)),
        ],
        out_specs=pl.BlockSpec((_B, d), lambda i: (0, 0)),
        out_shape=jax.ShapeDtypeStruct((_B, d), jnp.float32),
    )(rank, bcol, counts, logits_p)

    return out
